# Initial kernel scaffold; baseline (speedup 1.0000x reference)
import jax
import jax.numpy as jnp
from jax.experimental import pallas as pl


def kernel(q, kv, edge_index, W_k, b_k, W_v, b_v, W_o, b_o):
    raise NotImplementedError("write your pallas kernel here")


# speedup vs baseline: 13.2034x; 13.2034x over previous
import functools

import jax
import jax.numpy as jnp
from jax import lax
from jax.experimental import pallas as pl
from jax.experimental.pallas import tpu as pltpu
from jax.experimental.pallas import tpu_sc as plsc

N = 10000
E = 320000
DIM_H = 128
NUM_HEADS = 8
HEAD_DIM = 16
NC = 2
NS = 16
NW = NC * NS
EDGES_PER_W = E // NW
CB = 80
NCHUNK = EDGES_PER_W // CB
ROWS_PER_TILE = N // NS
PROJ_B = 400


def _proj_body(x_ref, wk_ref, bk_ref, wv_ref, bv_ref, k_ref, v_ref):
    x = x_ref[...]
    cdims = (((1,), (1,)), ((), ()))
    k_ref[...] = lax.dot_general(x, wk_ref[...], cdims,
                                 preferred_element_type=jnp.float32) + bk_ref[...]
    v_ref[...] = lax.dot_general(x, wv_ref[...], cdims,
                                 preferred_element_type=jnp.float32) + bv_ref[...]


def _project_kv(kv, W_k, b_k, W_v, b_v):
    return pl.pallas_call(
        _proj_body,
        grid=(N // PROJ_B,),
        in_specs=[
            pl.BlockSpec((PROJ_B, DIM_H), lambda i: (i, 0)),
            pl.BlockSpec((DIM_H, DIM_H), lambda i: (0, 0)),
            pl.BlockSpec((1, DIM_H), lambda i: (0, 0)),
            pl.BlockSpec((DIM_H, DIM_H), lambda i: (0, 0)),
            pl.BlockSpec((1, DIM_H), lambda i: (0, 0)),
        ],
        out_specs=[
            pl.BlockSpec((PROJ_B, DIM_H), lambda i: (i, 0)),
            pl.BlockSpec((PROJ_B, DIM_H), lambda i: (i, 0)),
        ],
        out_shape=[
            jax.ShapeDtypeStruct((N, DIM_H), jnp.float32),
            jax.ShapeDtypeStruct((N, DIM_H), jnp.float32),
        ],
    )(kv, W_k, b_k.reshape(1, DIM_H), W_v, b_v.reshape(1, DIM_H))


def _sc_edge_body(k_hbm, v_hbm, q_hbm, ei_hbm, outa_hbm, outd_hbm,
                  agg_sh, den_sh, src_v, dst_v, k_v, q_v, vmsg_v, wbuf2d,
                  sem_k, sem_q, sem_v):
    cid = lax.axis_index("c")
    sid = lax.axis_index("s")
    wid = sid * NC + cid

    zero16 = jnp.zeros((16,), jnp.float32)

    def _zrow(r, carry):
        for j in range(DIM_H // 16):
            vmsg_v[r, pl.ds(j * 16, 16)] = zero16
        return carry

    lax.fori_loop(0, CB, _zrow, 0)

    lane = lax.iota(jnp.int32, 16)

    def _zwrow(r, carry):
        flat = r * 16 + lane
        plsc.store_scatter(wbuf2d, [flat // NUM_HEADS, flat % NUM_HEADS],
                           zero16)
        return carry

    lax.fori_loop(0, CB * NUM_HEADS // 16, _zwrow, 0)

    row0 = sid * ROWS_PER_TILE
    nfull = ROWS_PER_TILE // CB
    rem = ROWS_PER_TILE - nfull * CB

    def _zagg(i, carry):
        r = row0 + i * CB
        pltpu.sync_copy(vmsg_v, agg_sh.at[pl.ds(r, CB), :])
        pltpu.sync_copy(wbuf2d, den_sh.at[pl.ds(r, CB), :])
        return carry

    lax.fori_loop(0, nfull, _zagg, 0)
    if rem:
        r = row0 + nfull * CB
        pltpu.sync_copy(vmsg_v.at[pl.ds(0, rem), :],
                        agg_sh.at[pl.ds(r, rem), :])
        pltpu.sync_copy(wbuf2d.at[pl.ds(0, rem), :],
                        den_sh.at[pl.ds(r, rem), :])
    plsc.subcore_barrier()

    ebase = wid * EDGES_PER_W

    def _chunk(ci, carry):
        base = ebase + ci * CB
        pltpu.sync_copy(ei_hbm.at[0, pl.ds(base, CB)], src_v)
        pltpu.sync_copy(ei_hbm.at[1, pl.ds(base, CB)], dst_v)
        ck = pltpu.async_copy(k_hbm.at[src_v], k_v, sem_k)
        cq = pltpu.async_copy(q_hbm.at[dst_v], q_v, sem_q)
        cv = pltpu.async_copy(v_hbm.at[src_v], vmsg_v, sem_v)
        ck.wait()
        cq.wait()
        cv.wait()

        def _group(g, gcarry):
            e_ids = lane + g * 16
            for h in range(NUM_HEADS):
                acc = jnp.zeros((16,), jnp.float32)
                for d in range(HEAD_DIM):
                    col = jnp.full((16,), h * HEAD_DIM + d, jnp.int32)
                    kvals = plsc.load_gather(k_v, [e_ids, col])
                    qvals = plsc.load_gather(q_v, [e_ids, col])
                    acc = acc + kvals * qvals
                w = jnp.exp(acc * 0.25)
                plsc.store_scatter(
                    wbuf2d, [e_ids, jnp.full((16,), h, jnp.int32)], w)
                for d in range(HEAD_DIM):
                    col = jnp.full((16,), h * HEAD_DIM + d, jnp.int32)
                    vvals = plsc.load_gather(vmsg_v, [e_ids, col])
                    plsc.store_scatter(vmsg_v, [e_ids, col], vvals * w)
            return gcarry

        lax.fori_loop(0, CB // 16, _group, 0)
        pltpu.sync_copy(vmsg_v, agg_sh.at[dst_v], add=True)
        pltpu.sync_copy(wbuf2d, den_sh.at[dst_v], add=True)
        return carry

    lax.fori_loop(0, NCHUNK, _chunk, 0)
    plsc.subcore_barrier()
    pltpu.sync_copy(agg_sh.at[pl.ds(row0, ROWS_PER_TILE), :],
                    outa_hbm.at[cid, pl.ds(row0, ROWS_PER_TILE), :])
    pltpu.sync_copy(den_sh.at[pl.ds(row0, ROWS_PER_TILE), :],
                    outd_hbm.at[cid, pl.ds(row0, ROWS_PER_TILE), :])


_sc_edge = pl.kernel(
    _sc_edge_body,
    out_type=[
        jax.ShapeDtypeStruct((NC, N, DIM_H), jnp.float32),
        jax.ShapeDtypeStruct((NC, N, NUM_HEADS), jnp.float32),
    ],
    mesh=plsc.VectorSubcoreMesh(core_axis_name="c", subcore_axis_name="s",
                                num_cores=NC, num_subcores=NS),
    compiler_params=pltpu.CompilerParams(use_tc_tiling_on_sc=False,
                                         needs_layout_passes=False),
    scratch_types=[
        pltpu.VMEM_SHARED((N, DIM_H), jnp.float32),
        pltpu.VMEM_SHARED((N, NUM_HEADS), jnp.float32),
        pltpu.VMEM((CB,), jnp.int32),
        pltpu.VMEM((CB,), jnp.int32),
        pltpu.VMEM((CB, DIM_H), jnp.float32),
        pltpu.VMEM((CB, DIM_H), jnp.float32),
        pltpu.VMEM((CB, DIM_H), jnp.float32),
        pltpu.VMEM((CB, NUM_HEADS), jnp.float32),
        pltpu.SemaphoreType.DMA,
        pltpu.SemaphoreType.DMA,
        pltpu.SemaphoreType.DMA,
    ],
)


def _out_body(agg_ref, den_ref, wo_ref, bo_ref, o_ref):
    a = agg_ref[0] + agg_ref[1]
    dn = den_ref[0] + den_ref[1]
    rows = lax.broadcasted_iota(jnp.int32, (NUM_HEADS, DIM_H), 0)
    cols = lax.broadcasted_iota(jnp.int32, (NUM_HEADS, DIM_H), 1)
    sel = (rows == (cols // HEAD_DIM)).astype(jnp.float32)
    cdims_nt = (((1,), (0,)), ((), ()))
    den_w = lax.dot_general(dn, sel, cdims_nt,
                            preferred_element_type=jnp.float32) + 1e-16
    cdims_t = (((1,), (1,)), ((), ()))
    y = lax.dot_general(a / den_w, wo_ref[...], cdims_t,
                        preferred_element_type=jnp.float32) + bo_ref[...]
    o_ref[...] = y


def _out_proj(agg, den, W_o, b_o):
    return pl.pallas_call(
        _out_body,
        grid=(N // PROJ_B,),
        in_specs=[
            pl.BlockSpec((NC, PROJ_B, DIM_H), lambda i: (0, i, 0)),
            pl.BlockSpec((NC, PROJ_B, NUM_HEADS), lambda i: (0, i, 0)),
            pl.BlockSpec((DIM_H, DIM_H), lambda i: (0, 0)),
            pl.BlockSpec((1, DIM_H), lambda i: (0, 0)),
        ],
        out_specs=pl.BlockSpec((PROJ_B, DIM_H), lambda i: (i, 0)),
        out_shape=jax.ShapeDtypeStruct((N, DIM_H), jnp.float32),
    )(agg, den, W_o, b_o.reshape(1, DIM_H))


@jax.jit
def kernel(q, kv, edge_index, W_k, b_k, W_v, b_v, W_o, b_o):
    k_tab, v_tab = _project_kv(kv, W_k, b_k, W_v, b_v)
    agg, den = _sc_edge(k_tab, v_tab, q, edge_index)
    return _out_proj(agg, den, W_o, b_o)
